# baseline (speedup 1.0000x reference)
import functools
import jax
import jax.numpy as jnp
from jax import lax
from jax.experimental import pallas as pl
from jax.experimental.pallas import tpu as pltpu, tpu_sc as plsc

N = 10000
E = 320000
P = 65536
D = 128
DH = 256

NC, NS, L = 2, 16, 16
W = 128
EP2 = 327680
NROW = EP2 // W
RB = 32
EPAD = 720896
ECHUNK = 2048
HCH = 1024
NPAD = 10240
SINK = NPAD - 1

_mesh = plsc.VectorSubcoreMesh(core_axis_name="c", subcore_axis_name="s")
_sc_params = pltpu.CompilerParams(needs_layout_passes=False)


@functools.partial(
    pl.kernel,
    out_type=jax.ShapeDtypeStruct((32, 1, NPAD), jnp.float32),
    mesh=_mesh,
    compiler_params=_sc_params,
    scratch_types=[
        pltpu.VMEM((NPAD,), jnp.float32),
        pltpu.VMEM((2, ECHUNK), jnp.int32),
        pltpu.SemaphoreType.DMA,
        pltpu.SemaphoreType.DMA,
    ],
)
def _sc_degrees(s0, d0, s1, d1, dp, deg_v, idx_v, dsem0, dsem1):
    c = lax.axis_index("c")
    s = lax.axis_index("s")
    h = c * 2 + s // 8
    slot = s % 8
    use_dst = s >= 8
    dsems = (dsem0, dsem1)

    @pl.loop(0, NPAD // L)
    def _(i):
        deg_v[pl.ds(i * L, L)] = jnp.zeros((L,), jnp.float32)

    base = slot * (EP2 // 8)
    ones = jnp.ones((L,), jnp.float32)
    nch = EP2 // 8 // ECHUNK

    def fetch(t, k):
        off = base + t * ECHUNK

        @pl.when(jnp.logical_and(c == 0, jnp.logical_not(use_dst)))
        def _():
            pltpu.async_copy(s0.at[pl.ds(off, ECHUNK)], idx_v.at[k], dsems[k])

        @pl.when(jnp.logical_and(c == 0, use_dst))
        def _():
            pltpu.async_copy(d0.at[pl.ds(off, ECHUNK)], idx_v.at[k], dsems[k])

        @pl.when(jnp.logical_and(c == 1, jnp.logical_not(use_dst)))
        def _():
            pltpu.async_copy(s1.at[pl.ds(off, ECHUNK)], idx_v.at[k], dsems[k])

        @pl.when(jnp.logical_and(c == 1, use_dst))
        def _():
            pltpu.async_copy(d1.at[pl.ds(off, ECHUNK)], idx_v.at[k], dsems[k])

    fetch(0, 0)

    @pl.loop(0, nch // 2)
    def _(t2):
        for k in (0, 1):
            t = 2 * t2 + k
            pltpu.make_async_copy(s0.at[pl.ds(0, ECHUNK)], idx_v.at[k],
                                  dsems[k]).wait()

            @pl.when(t + 1 < nch)
            def _():
                fetch(t + 1, 1 - k)

            @pl.loop(0, ECHUNK // L)
            def _(kk):
                idx = idx_v[k, pl.ds(kk * L, L)]
                plsc.addupdate_scatter(deg_v, [idx], ones)

    pltpu.sync_copy(deg_v, dp.at[h * 8 + slot, 0])


@functools.partial(
    pl.kernel,
    out_type=jax.ShapeDtypeStruct((2, NPAD, D), jnp.float32),
    mesh=_mesh,
    compiler_params=_sc_params,
    scratch_types=[
        pltpu.VMEM((32, W), jnp.int32),
        pltpu.VMEM((32, W), jnp.int32),
        pltpu.VMEM((2, W, D), jnp.float32),
        pltpu.VMEM_SHARED((NPAD, D), jnp.float32),
        pltpu.SemaphoreType.DMA,
        pltpu.SemaphoreType.DMA,
    ],
)
def _sc_segsum(tt, s03, d03, s13, d13, ag, src_v, dst_v, rows, acc,
               sem0, sem1):
    c = lax.axis_index("c")
    s = lax.axis_index("s")
    nsub = NROW // NS

    zrow = rows.at[0]

    @pl.loop(0, W)
    def _(r):
        @pl.loop(0, D // L)
        def _(k):
            zrow[r, pl.ds(k * L, L)] = jnp.zeros((L,), jnp.float32)

    nrows_sub = NPAD // NS

    @pl.loop(0, nrows_sub // W)
    def _(t):
        pltpu.sync_copy(zrow, acc.at[pl.ds(s * nrows_sub + t * W, W)])

    plsc.subcore_barrier()
    sems = (sem0, sem1)

    @pl.loop(0, nsub // 32)
    def _(b):
        rbase = s * nsub + b * 32

        @pl.when(c == 0)
        def _():
            pltpu.sync_copy(s03.at[pl.ds(rbase, 32)], src_v)
            pltpu.sync_copy(d03.at[pl.ds(rbase, 32)], dst_v)

        @pl.when(c == 1)
        def _():
            pltpu.sync_copy(s13.at[pl.ds(rbase, 32)], src_v)
            pltpu.sync_copy(d13.at[pl.ds(rbase, 32)], dst_v)

        @pl.loop(0, 32)
        def _(j):
            @pl.loop(0, W // L)
            def _(k):
                src_v[j, pl.ds(k * L, L)] = src_v[j, pl.ds(k * L, L)] + c * NPAD

        for k in range(2):
            pltpu.async_copy(tt.at[src_v.at[k]], rows.at[k], sems[k])

        @pl.loop(0, 16)
        def _(t):
            for k in range(2):
                j = 2 * t + k
                pltpu.make_async_copy(tt.at[src_v.at[j]], rows.at[k],
                                      sems[k]).wait()
                pltpu.sync_copy(rows.at[k], acc.at[dst_v.at[j]], add=True)

                @pl.when(j + 2 < 32)
                def _():
                    pltpu.async_copy(tt.at[src_v.at[j + 2]], rows.at[k],
                                     sems[k])

    plsc.subcore_barrier()
    pltpu.sync_copy(acc.at[pl.ds(s * nrows_sub, nrows_sub)],
                    ag.at[c, pl.ds(s * nrows_sub, nrows_sub)])


@functools.partial(
    pl.kernel,
    out_type=jax.ShapeDtypeStruct((EPAD,), jnp.float32),
    mesh=_mesh,
    compiler_params=_sc_params,
    scratch_types=[
        pltpu.VMEM((NPAD,), jnp.float32),
        pltpu.VMEM((NPAD,), jnp.float32),
        pltpu.VMEM((2, HCH), jnp.int32),
        pltpu.VMEM((2, HCH), jnp.int32),
        pltpu.VMEM((2, HCH), jnp.float32),
        pltpu.VMEM((2, HCH), jnp.float32),
        pltpu.VMEM((L,), jnp.float32),
        pltpu.SemaphoreType.DMA,
        pltpu.SemaphoreType.DMA,
    ],
)
def _sc_head(a_hbm, b_hbm, idxa, idxb, mf, blv, out,
             a_v, b_v, ia_v, ib_v, m_v, o_v, bl_v, hsem0, hsem1):
    c = lax.axis_index("c")
    s = lax.axis_index("s")
    w = s * NC + c
    hsems = (hsem0, hsem1)
    pltpu.sync_copy(a_hbm, a_v)
    pltpu.sync_copy(b_hbm, b_v)
    pltpu.sync_copy(blv, bl_v)
    bl = bl_v[...]
    base = w * (EPAD // (NC * NS))
    nch = EPAD // (NC * NS) // HCH

    def fetch(t, k):
        off = base + t * HCH
        pltpu.async_copy(idxa.at[pl.ds(off, HCH)], ia_v.at[k], hsems[k])
        pltpu.async_copy(idxb.at[pl.ds(off, HCH)], ib_v.at[k], hsems[k])
        pltpu.async_copy(mf.at[pl.ds(off, HCH)], m_v.at[k], hsems[k])

    fetch(0, 0)

    @pl.loop(0, nch // 2)
    def _(t2):
        for k in (0, 1):
            t = 2 * t2 + k
            off = base + t * HCH
            pltpu.make_async_copy(idxa.at[pl.ds(0, HCH)], ia_v.at[k],
                                  hsems[k]).wait()
            pltpu.make_async_copy(idxa.at[pl.ds(0, HCH)], ib_v.at[k],
                                  hsems[k]).wait()
            pltpu.make_async_copy(mf.at[pl.ds(0, HCH)], m_v.at[k],
                                  hsems[k]).wait()

            @pl.when(t + 1 < nch)
            def _():
                fetch(t + 1, 1 - k)

            @pl.loop(0, HCH // L)
            def _(kk):
                ia = ia_v[k, pl.ds(kk * L, L)]
                ib = ib_v[k, pl.ds(kk * L, L)]
                ga = plsc.load_gather(a_v, [ia])
                gb = plsc.load_gather(b_v, [ib])
                z = m_v[k, pl.ds(kk * L, L)] * (ga + gb) + bl
                o_v[k, pl.ds(kk * L, L)] = 1.0 / (1.0 + jnp.exp(-z))

            pltpu.sync_copy(o_v.at[k], out.at[pl.ds(off, HCH)])


_TCB = 1024


def _tc_prep_body(dp_ref, x_ref, g_ref, nor_ref):
    dp = dp_ref[...].reshape(4, 8, _TCB)
    deg = jnp.sum(dp, axis=1)
    nr = lax.rsqrt(jnp.maximum(deg, 1.0))
    ns0, nd0, ns1, nd1 = nr[0], nr[1], nr[2], nr[3]
    x = x_ref[...]
    g_ref[0] = x * ns0[:, None]
    g_ref[1] = x * ns1[:, None]
    nor_ref[...] = jnp.stack([nd0, nd1, ns0, ns1], axis=1)


def _tc_prep(dp, xp):
    return pl.pallas_call(
        _tc_prep_body,
        grid=(NPAD // _TCB,),
        in_specs=[
            pl.BlockSpec((32, 1, _TCB), lambda g: (0, 0, g)),
            pl.BlockSpec((_TCB, D), lambda g: (g, 0)),
        ],
        out_specs=[
            pl.BlockSpec((2, _TCB, D), lambda g: (0, g, 0)),
            pl.BlockSpec((_TCB, 4), lambda g: (g, 0)),
        ],
        out_shape=[
            jax.ShapeDtypeStruct((2, NPAD, D), jnp.float32),
            jax.ShapeDtypeStruct((NPAD, 4), jnp.float32),
        ],
    )(dp, xp)


def _tc_layer1_body(ag_ref, nor_ref, w10_ref, w11_ref, b10_ref, b11_ref,
                    w20_ref, w21_ref, pp_ref):
    nor = nor_ref[...]
    nd0, nd1, ns0, ns1 = nor[:, 0], nor[:, 1], nor[:, 2], nor[:, 3]
    t0 = ag_ref[0] * nd0[:, None]
    t1 = ag_ref[1] * nd1[:, None]
    hpre = (jnp.dot(t0, w10_ref[...], preferred_element_type=jnp.float32)
            + jnp.dot(t1, w11_ref[...], preferred_element_type=jnp.float32)
            + b10_ref[...] + b11_ref[...]) * 0.5
    h = jnp.maximum(hpre, 0.0)
    pp_ref[0] = jnp.dot(h * ns0[:, None], w20_ref[...],
                        preferred_element_type=jnp.float32)
    pp_ref[1] = jnp.dot(h * ns1[:, None], w21_ref[...],
                        preferred_element_type=jnp.float32)


def _tc_layer1(ag, nor, w10, w11, b10, b11, w20, w21):
    full = lambda shape: pl.BlockSpec(shape, lambda g: tuple(0 for _ in shape))
    return pl.pallas_call(
        _tc_layer1_body,
        grid=(NPAD // _TCB,),
        in_specs=[
            pl.BlockSpec((2, _TCB, D), lambda g: (0, g, 0)),
            pl.BlockSpec((_TCB, 4), lambda g: (g, 0)),
            full((D, DH)), full((D, DH)),
            full((1, DH)), full((1, DH)),
            full((DH, D)), full((DH, D)),
        ],
        out_specs=pl.BlockSpec((2, _TCB, D), lambda g: (0, g, 0)),
        out_shape=jax.ShapeDtypeStruct((2, NPAD, D), jnp.float32),
    )(ag, nor, w10, w11, b10, b11, w20, w21)


def _tc_layer2_body(ag_ref, nor_ref, b20_ref, b21_ref, wl_ref, ab_ref):
    nor = nor_ref[...]
    nd0, nd1 = nor[:, 0], nor[:, 1]
    h2 = (ag_ref[0] * nd0[:, None] + ag_ref[1] * nd1[:, None]
          + b20_ref[...] + b21_ref[...]) * 0.5
    r2 = jnp.maximum(h2, 0.0)
    wl = wl_ref[...]
    a = jnp.sum(r2 * wl[0][None, :], axis=1)
    b = jnp.sum(r2 * wl[1][None, :], axis=1)
    ab_ref[...] = jnp.stack([a, b], axis=1)


def _tc_layer2(ag, nor, b20, b21, wl2):
    full = lambda shape: pl.BlockSpec(shape, lambda g: tuple(0 for _ in shape))
    return pl.pallas_call(
        _tc_layer2_body,
        grid=(NPAD // _TCB,),
        in_specs=[
            pl.BlockSpec((2, _TCB, D), lambda g: (0, g, 0)),
            pl.BlockSpec((_TCB, 4), lambda g: (g, 0)),
            full((1, D)), full((1, D)), full((2, D)),
        ],
        out_specs=pl.BlockSpec((_TCB, 2), lambda g: (g, 0)),
        out_shape=jax.ShapeDtypeStruct((NPAD, 2), jnp.float32),
    )(ag, nor, b20, b21, wl2)


def kernel(x, edge_index_rel0, edge_index_rel1, mask_rel0, mask_rel1, n_pairs,
           W1_rel0, W1_rel1, b1_rel0, b1_rel1,
           W2_rel0, W2_rel1, b2_rel0, b2_rel1, Wl, bl):
    src0, dst0 = edge_index_rel0[0], edge_index_rel0[1]
    src1, dst1 = edge_index_rel1[0], edge_index_rel1[1]
    sinkpad = jnp.full((EP2 - E,), SINK, jnp.int32)
    s0p = jnp.concatenate([src0, sinkpad])
    d0p = jnp.concatenate([dst0, sinkpad])
    s1p = jnp.concatenate([src1, sinkpad])
    d1p = jnp.concatenate([dst1, sinkpad])
    s03, d03 = s0p.reshape(NROW, W), d0p.reshape(NROW, W)
    s13, d13 = s1p.reshape(NROW, W), d1p.reshape(NROW, W)

    dp = _sc_degrees(s0p, d0p, s1p, d1p)
    xp = jnp.pad(x, ((0, NPAD - N), (0, 0)))
    g, nor = _tc_prep(dp, xp)

    ag1 = _sc_segsum(g.reshape(2 * NPAD, D), s03, d03, s13, d13)
    pp = _tc_layer1(ag1, nor,
                    W1_rel0, W1_rel1,
                    b1_rel0.reshape(1, DH), b1_rel1.reshape(1, DH),
                    W2_rel0, W2_rel1)
    ag2 = _sc_segsum(pp.reshape(2 * NPAD, D), s03, d03, s13, d13)
    ab = _tc_layer2(ag2, nor,
                    b2_rel0.reshape(1, D), b2_rel1.reshape(1, D),
                    Wl.reshape(2, D))
    a_n, b_n = ab[:, 0], ab[:, 1]

    pad = EPAD - (2 * E + P)
    idxa = jnp.concatenate([src0, src1, n_pairs[:, 0],
                            jnp.zeros((pad,), jnp.int32)])
    idxb = jnp.concatenate([dst0, dst1, n_pairs[:, 1],
                            jnp.zeros((pad,), jnp.int32)])
    mf = jnp.concatenate([mask_rel0.astype(jnp.float32),
                          mask_rel1.astype(jnp.float32),
                          jnp.ones((P,), jnp.float32),
                          jnp.zeros((pad,), jnp.float32)])
    blv = jnp.broadcast_to(bl.astype(jnp.float32).reshape(-1)[:1], (L,))

    out = _sc_head(a_n, b_n, idxa, idxb, mf, blv)
    return out[:2 * E + P, None]


# speedup vs baseline: 1.0224x; 1.0224x over previous
import functools
import jax
import jax.numpy as jnp
from jax import lax
from jax.experimental import pallas as pl
from jax.experimental.pallas import tpu as pltpu, tpu_sc as plsc

N = 10000
E = 320000
P = 65536
D = 128
DH = 256

NC, NS, L = 2, 16, 16
W = 80
EP2 = 327680
NROW = EP2 // W
RB = 32
EPAD = 720896
ECHUNK = 2048
HCH = 1024
NPAD = 10240
SINK = NPAD - 1

_mesh = plsc.VectorSubcoreMesh(core_axis_name="c", subcore_axis_name="s")
_sc_params = pltpu.CompilerParams(needs_layout_passes=False)


@functools.partial(
    pl.kernel,
    out_type=jax.ShapeDtypeStruct((32, 1, NPAD), jnp.float32),
    mesh=_mesh,
    compiler_params=_sc_params,
    scratch_types=[
        pltpu.VMEM((NPAD,), jnp.float32),
        pltpu.VMEM((2, ECHUNK), jnp.int32),
        pltpu.SemaphoreType.DMA,
        pltpu.SemaphoreType.DMA,
    ],
)
def _sc_degrees(s0, d0, s1, d1, dp, deg_v, idx_v, dsem0, dsem1):
    c = lax.axis_index("c")
    s = lax.axis_index("s")
    h = c * 2 + s // 8
    slot = s % 8
    use_dst = s >= 8
    dsems = (dsem0, dsem1)

    @pl.loop(0, NPAD // L)
    def _(i):
        deg_v[pl.ds(i * L, L)] = jnp.zeros((L,), jnp.float32)

    base = slot * (EP2 // 8)
    ones = jnp.ones((L,), jnp.float32)
    nch = EP2 // 8 // ECHUNK

    def fetch(t, k):
        off = base + t * ECHUNK

        @pl.when(jnp.logical_and(c == 0, jnp.logical_not(use_dst)))
        def _():
            pltpu.async_copy(s0.at[pl.ds(off, ECHUNK)], idx_v.at[k], dsems[k])

        @pl.when(jnp.logical_and(c == 0, use_dst))
        def _():
            pltpu.async_copy(d0.at[pl.ds(off, ECHUNK)], idx_v.at[k], dsems[k])

        @pl.when(jnp.logical_and(c == 1, jnp.logical_not(use_dst)))
        def _():
            pltpu.async_copy(s1.at[pl.ds(off, ECHUNK)], idx_v.at[k], dsems[k])

        @pl.when(jnp.logical_and(c == 1, use_dst))
        def _():
            pltpu.async_copy(d1.at[pl.ds(off, ECHUNK)], idx_v.at[k], dsems[k])

    fetch(0, 0)

    @pl.loop(0, nch // 2)
    def _(t2):
        for k in (0, 1):
            t = 2 * t2 + k
            pltpu.make_async_copy(s0.at[pl.ds(0, ECHUNK)], idx_v.at[k],
                                  dsems[k]).wait()

            @pl.when(t + 1 < nch)
            def _():
                fetch(t + 1, 1 - k)

            @pl.loop(0, ECHUNK // L)
            def _(kk):
                idx = idx_v[k, pl.ds(kk * L, L)]
                plsc.addupdate_scatter(deg_v, [idx], ones)

    pltpu.sync_copy(deg_v, dp.at[h * 8 + slot, 0])


@functools.partial(
    pl.kernel,
    out_type=jax.ShapeDtypeStruct((2, NPAD, D), jnp.float32),
    mesh=_mesh,
    compiler_params=_sc_params,
    scratch_types=[
        pltpu.VMEM((32, W), jnp.int32),
        pltpu.VMEM((32, W), jnp.int32),
        pltpu.VMEM((4, W, D), jnp.float32),
        pltpu.VMEM_SHARED((NPAD, D), jnp.float32),
        pltpu.SemaphoreType.DMA,
        pltpu.SemaphoreType.DMA,
        pltpu.SemaphoreType.DMA,
        pltpu.SemaphoreType.DMA,
    ],
)
def _sc_segsum(tt, s03, d03, s13, d13, ag, src_v, dst_v, rows, acc,
               sem0, sem1, sem2, sem3):
    c = lax.axis_index("c")
    s = lax.axis_index("s")
    nsub = NROW // NS

    zrow = rows.at[0]

    @pl.loop(0, W)
    def _(r):
        @pl.loop(0, D // L)
        def _(k):
            zrow[r, pl.ds(k * L, L)] = jnp.zeros((L,), jnp.float32)

    nrows_sub = NPAD // NS

    @pl.loop(0, nrows_sub // W)
    def _(t):
        pltpu.sync_copy(zrow, acc.at[pl.ds(s * nrows_sub + t * W, W)])

    plsc.subcore_barrier()
    sems = (sem0, sem1, sem2, sem3)

    @pl.loop(0, nsub // 32)
    def _(b):
        rbase = s * nsub + b * 32

        @pl.when(c == 0)
        def _():
            pltpu.sync_copy(s03.at[pl.ds(rbase, 32)], src_v)
            pltpu.sync_copy(d03.at[pl.ds(rbase, 32)], dst_v)

        @pl.when(c == 1)
        def _():
            pltpu.sync_copy(s13.at[pl.ds(rbase, 32)], src_v)
            pltpu.sync_copy(d13.at[pl.ds(rbase, 32)], dst_v)

        @pl.loop(0, 32)
        def _(j):
            @pl.loop(0, W // L)
            def _(k):
                src_v[j, pl.ds(k * L, L)] = src_v[j, pl.ds(k * L, L)] + c * NPAD

        for k in range(4):
            pltpu.async_copy(tt.at[src_v.at[k]], rows.at[k], sems[k])

        @pl.loop(0, 8)
        def _(t):
            for k in range(4):
                j = 4 * t + k
                pltpu.make_async_copy(tt.at[src_v.at[j]], rows.at[k],
                                      sems[k]).wait()
                pltpu.sync_copy(rows.at[k], acc.at[dst_v.at[j]], add=True)

                @pl.when(j + 4 < 32)
                def _():
                    pltpu.async_copy(tt.at[src_v.at[j + 4]], rows.at[k],
                                     sems[k])

    plsc.subcore_barrier()
    pltpu.sync_copy(acc.at[pl.ds(s * nrows_sub, nrows_sub)],
                    ag.at[c, pl.ds(s * nrows_sub, nrows_sub)])


@functools.partial(
    pl.kernel,
    out_type=jax.ShapeDtypeStruct((EPAD,), jnp.float32),
    mesh=_mesh,
    compiler_params=_sc_params,
    scratch_types=[
        pltpu.VMEM((NPAD,), jnp.float32),
        pltpu.VMEM((NPAD,), jnp.float32),
        pltpu.VMEM((2, HCH), jnp.int32),
        pltpu.VMEM((2, HCH), jnp.int32),
        pltpu.VMEM((2, HCH), jnp.float32),
        pltpu.VMEM((2, HCH), jnp.float32),
        pltpu.VMEM((L,), jnp.float32),
        pltpu.SemaphoreType.DMA,
        pltpu.SemaphoreType.DMA,
    ],
)
def _sc_head(a_hbm, b_hbm, idxa, idxb, mf, blv, out,
             a_v, b_v, ia_v, ib_v, m_v, o_v, bl_v, hsem0, hsem1):
    c = lax.axis_index("c")
    s = lax.axis_index("s")
    w = s * NC + c
    hsems = (hsem0, hsem1)
    pltpu.sync_copy(a_hbm, a_v)
    pltpu.sync_copy(b_hbm, b_v)
    pltpu.sync_copy(blv, bl_v)
    bl = bl_v[...]
    base = w * (EPAD // (NC * NS))
    nch = EPAD // (NC * NS) // HCH

    def fetch(t, k):
        off = base + t * HCH
        pltpu.async_copy(idxa.at[pl.ds(off, HCH)], ia_v.at[k], hsems[k])
        pltpu.async_copy(idxb.at[pl.ds(off, HCH)], ib_v.at[k], hsems[k])
        pltpu.async_copy(mf.at[pl.ds(off, HCH)], m_v.at[k], hsems[k])

    fetch(0, 0)

    @pl.loop(0, nch // 2)
    def _(t2):
        for k in (0, 1):
            t = 2 * t2 + k
            off = base + t * HCH
            pltpu.make_async_copy(idxa.at[pl.ds(0, HCH)], ia_v.at[k],
                                  hsems[k]).wait()
            pltpu.make_async_copy(idxa.at[pl.ds(0, HCH)], ib_v.at[k],
                                  hsems[k]).wait()
            pltpu.make_async_copy(mf.at[pl.ds(0, HCH)], m_v.at[k],
                                  hsems[k]).wait()

            @pl.when(t + 1 < nch)
            def _():
                fetch(t + 1, 1 - k)

            @pl.loop(0, HCH // L)
            def _(kk):
                ia = ia_v[k, pl.ds(kk * L, L)]
                ib = ib_v[k, pl.ds(kk * L, L)]
                ga = plsc.load_gather(a_v, [ia])
                gb = plsc.load_gather(b_v, [ib])
                z = m_v[k, pl.ds(kk * L, L)] * (ga + gb) + bl
                o_v[k, pl.ds(kk * L, L)] = 1.0 / (1.0 + jnp.exp(-z))

            pltpu.sync_copy(o_v.at[k], out.at[pl.ds(off, HCH)])


_TCB = 1024


def _tc_prep_body(dp_ref, x_ref, g_ref, nor_ref):
    dp = dp_ref[...].reshape(4, 8, _TCB)
    deg = jnp.sum(dp, axis=1)
    nr = lax.rsqrt(jnp.maximum(deg, 1.0))
    ns0, nd0, ns1, nd1 = nr[0], nr[1], nr[2], nr[3]
    x = x_ref[...]
    g_ref[0] = x * ns0[:, None]
    g_ref[1] = x * ns1[:, None]
    nor_ref[...] = jnp.stack([nd0, nd1, ns0, ns1], axis=1)


def _tc_prep(dp, xp):
    return pl.pallas_call(
        _tc_prep_body,
        grid=(NPAD // _TCB,),
        in_specs=[
            pl.BlockSpec((32, 1, _TCB), lambda g: (0, 0, g)),
            pl.BlockSpec((_TCB, D), lambda g: (g, 0)),
        ],
        out_specs=[
            pl.BlockSpec((2, _TCB, D), lambda g: (0, g, 0)),
            pl.BlockSpec((_TCB, 4), lambda g: (g, 0)),
        ],
        out_shape=[
            jax.ShapeDtypeStruct((2, NPAD, D), jnp.float32),
            jax.ShapeDtypeStruct((NPAD, 4), jnp.float32),
        ],
    )(dp, xp)


def _tc_layer1_body(ag_ref, nor_ref, w10_ref, w11_ref, b10_ref, b11_ref,
                    w20_ref, w21_ref, pp_ref):
    nor = nor_ref[...]
    nd0, nd1, ns0, ns1 = nor[:, 0], nor[:, 1], nor[:, 2], nor[:, 3]
    t0 = ag_ref[0] * nd0[:, None]
    t1 = ag_ref[1] * nd1[:, None]
    hpre = (jnp.dot(t0, w10_ref[...], preferred_element_type=jnp.float32)
            + jnp.dot(t1, w11_ref[...], preferred_element_type=jnp.float32)
            + b10_ref[...] + b11_ref[...]) * 0.5
    h = jnp.maximum(hpre, 0.0)
    pp_ref[0] = jnp.dot(h * ns0[:, None], w20_ref[...],
                        preferred_element_type=jnp.float32)
    pp_ref[1] = jnp.dot(h * ns1[:, None], w21_ref[...],
                        preferred_element_type=jnp.float32)


def _tc_layer1(ag, nor, w10, w11, b10, b11, w20, w21):
    full = lambda shape: pl.BlockSpec(shape, lambda g: tuple(0 for _ in shape))
    return pl.pallas_call(
        _tc_layer1_body,
        grid=(NPAD // _TCB,),
        in_specs=[
            pl.BlockSpec((2, _TCB, D), lambda g: (0, g, 0)),
            pl.BlockSpec((_TCB, 4), lambda g: (g, 0)),
            full((D, DH)), full((D, DH)),
            full((1, DH)), full((1, DH)),
            full((DH, D)), full((DH, D)),
        ],
        out_specs=pl.BlockSpec((2, _TCB, D), lambda g: (0, g, 0)),
        out_shape=jax.ShapeDtypeStruct((2, NPAD, D), jnp.float32),
    )(ag, nor, w10, w11, b10, b11, w20, w21)


def _tc_layer2_body(ag_ref, nor_ref, b20_ref, b21_ref, wl_ref, ab_ref):
    nor = nor_ref[...]
    nd0, nd1 = nor[:, 0], nor[:, 1]
    h2 = (ag_ref[0] * nd0[:, None] + ag_ref[1] * nd1[:, None]
          + b20_ref[...] + b21_ref[...]) * 0.5
    r2 = jnp.maximum(h2, 0.0)
    wl = wl_ref[...]
    a = jnp.sum(r2 * wl[0][None, :], axis=1)
    b = jnp.sum(r2 * wl[1][None, :], axis=1)
    ab_ref[...] = jnp.stack([a, b], axis=1)


def _tc_layer2(ag, nor, b20, b21, wl2):
    full = lambda shape: pl.BlockSpec(shape, lambda g: tuple(0 for _ in shape))
    return pl.pallas_call(
        _tc_layer2_body,
        grid=(NPAD // _TCB,),
        in_specs=[
            pl.BlockSpec((2, _TCB, D), lambda g: (0, g, 0)),
            pl.BlockSpec((_TCB, 4), lambda g: (g, 0)),
            full((1, D)), full((1, D)), full((2, D)),
        ],
        out_specs=pl.BlockSpec((_TCB, 2), lambda g: (g, 0)),
        out_shape=jax.ShapeDtypeStruct((NPAD, 2), jnp.float32),
    )(ag, nor, b20, b21, wl2)


def kernel(x, edge_index_rel0, edge_index_rel1, mask_rel0, mask_rel1, n_pairs,
           W1_rel0, W1_rel1, b1_rel0, b1_rel1,
           W2_rel0, W2_rel1, b2_rel0, b2_rel1, Wl, bl):
    src0, dst0 = edge_index_rel0[0], edge_index_rel0[1]
    src1, dst1 = edge_index_rel1[0], edge_index_rel1[1]
    sinkpad = jnp.full((EP2 - E,), SINK, jnp.int32)
    s0p = jnp.concatenate([src0, sinkpad])
    d0p = jnp.concatenate([dst0, sinkpad])
    s1p = jnp.concatenate([src1, sinkpad])
    d1p = jnp.concatenate([dst1, sinkpad])
    s03, d03 = s0p.reshape(NROW, W), d0p.reshape(NROW, W)
    s13, d13 = s1p.reshape(NROW, W), d1p.reshape(NROW, W)

    dp = _sc_degrees(s0p, d0p, s1p, d1p)
    xp = jnp.pad(x, ((0, NPAD - N), (0, 0)))
    g, nor = _tc_prep(dp, xp)

    ag1 = _sc_segsum(g.reshape(2 * NPAD, D), s03, d03, s13, d13)
    pp = _tc_layer1(ag1, nor,
                    W1_rel0, W1_rel1,
                    b1_rel0.reshape(1, DH), b1_rel1.reshape(1, DH),
                    W2_rel0, W2_rel1)
    ag2 = _sc_segsum(pp.reshape(2 * NPAD, D), s03, d03, s13, d13)
    ab = _tc_layer2(ag2, nor,
                    b2_rel0.reshape(1, D), b2_rel1.reshape(1, D),
                    Wl.reshape(2, D))
    a_n, b_n = ab[:, 0], ab[:, 1]

    pad = EPAD - (2 * E + P)
    idxa = jnp.concatenate([src0, src1, n_pairs[:, 0],
                            jnp.zeros((pad,), jnp.int32)])
    idxb = jnp.concatenate([dst0, dst1, n_pairs[:, 1],
                            jnp.zeros((pad,), jnp.int32)])
    mf = jnp.concatenate([mask_rel0.astype(jnp.float32),
                          mask_rel1.astype(jnp.float32),
                          jnp.ones((P,), jnp.float32),
                          jnp.zeros((pad,), jnp.float32)])
    blv = jnp.broadcast_to(bl.astype(jnp.float32).reshape(-1)[:1], (L,))

    out = _sc_head(a_n, b_n, idxa, idxb, mf, blv)
    return out[:2 * E + P, None]
